# Initial kernel scaffold; baseline (speedup 1.0000x reference)
import jax
import jax.numpy as jnp
from jax.experimental import pallas as pl


def kernel(x, edge_index, edge_weight, W1, b1, W2, b2):
    raise NotImplementedError("write your pallas kernel here")


# speedup vs baseline: 4.0165x; 4.0165x over previous
import functools

import jax
import jax.numpy as jnp
from jax import lax
from jax.experimental import pallas as pl
from jax.experimental.pallas import tpu as pltpu
from jax.experimental.pallas import tpu_sc as plsc

N = 10000
E = 320000
D = 128
DP = 144
NW = 32
EPT = E // NW
C = 80
CHUNKS = EPT // C
RPT = N // 16

_mesh = plsc.VectorSubcoreMesh(core_axis_name="c", subcore_axis_name="s")


@functools.partial(
    pl.kernel,
    out_type=jax.ShapeDtypeStruct((2, N, DP), jnp.float32),
    mesh=_mesh,
    scratch_types=[
        pltpu.VMEM((C,), jnp.int32),
        pltpu.VMEM((C,), jnp.int32),
        pltpu.VMEM((C,), jnp.float32),
        pltpu.VMEM((C, DP), jnp.float32),
        pltpu.VMEM_SHARED((N, DP), jnp.float32),
        pltpu.SemaphoreType.DMA,
    ],
    compiler_params=pltpu.CompilerParams(use_tc_tiling_on_sc=False,
                                         needs_layout_passes=False),
)
def _sc_agg(x_hbm, src_hbm, dst_hbm, w_hbm, out_hbm,
            src_v, dst_v, w_v, rows_v, acc, sem):
    cid = lax.axis_index("c")
    sid = lax.axis_index("s")
    wid = cid * 16 + sid

    zeros16 = jnp.zeros((16,), jnp.float32)

    def _zero_body(t, _):
        i = t // (DP // 16)
        j = (t % (DP // 16)) * 16
        rows_v[i, pl.ds(j, 16)] = zeros16
        return _
    lax.fori_loop(0, C * (DP // 16), _zero_body, None)

    r0 = sid * RPT
    for k in range((RPT + C - 1) // C):
        nrows = min(C, RPT - k * C)
        pltpu.sync_copy(rows_v.at[pl.ds(0, nrows)],
                        acc.at[pl.ds(r0 + k * C, nrows)])
    plsc.subcore_barrier()

    iota16 = lax.iota(jnp.int32, 16)
    ebase = wid * EPT

    def _chunk(g, _):
        base = ebase + g * C
        pltpu.sync_copy(src_hbm.at[pl.ds(base, C)], src_v)
        pltpu.sync_copy(dst_hbm.at[pl.ds(base, C)], dst_v)
        pltpu.sync_copy(w_hbm.at[pl.ds(base, C)], w_v)
        pltpu.async_copy(x_hbm.at[src_v], rows_v, sem).wait()
        for e in range(C):
            if e % 16 == 0:
                w_blk = w_v[pl.ds(e, 16)]
            w_s = jnp.sum(jnp.where(iota16 == (e % 16), w_blk, 0.0))
            wspl = jnp.full((16,), w_s, jnp.float32)
            for j in range(D // 16):
                sl = pl.ds(j * 16, 16)
                rows_v[e, sl] = rows_v[e, sl] * wspl
            ind = jnp.where(wspl != 0.0, 1.0, 0.0).astype(jnp.float32)
            sp = jnp.where(iota16 == 0, wspl,
                           jnp.where(iota16 == 1, ind, 0.0)).astype(jnp.float32)
            rows_v[e, pl.ds(D, 16)] = sp
        pltpu.sync_copy(rows_v, acc.at[dst_v], add=True)
        return _
    lax.fori_loop(0, CHUNKS, _chunk, None)

    plsc.subcore_barrier()

    pltpu.sync_copy(acc.at[pl.ds(r0, RPT)], out_hbm.at[cid, pl.ds(r0, RPT)])


def _combine_body(p_ref, o_ref):
    s = p_ref[0] + p_ref[1]
    cnt = s[:, 129:130]
    o_ref[...] = s / cnt


def _final_body(p_ref, w1_ref, b1_ref, w2_ref, b2_ref, o_ref):
    s = p_ref[0] + p_ref[1]
    cnt = s[:, 129:130]
    r = s[:, 128:129] / cnt
    bf = s[:, :D] / cnt
    w21 = lax.dot_general(w2_ref[...], w1_ref[...],
                          (((1,), (0,)), ((), ())),
                          preferred_element_type=jnp.float32,
                          precision=lax.Precision.HIGHEST)
    u = lax.dot_general(b1_ref[...], w2_ref[...],
                        (((1,), (1,)), ((), ())),
                        preferred_element_type=jnp.float32,
                        precision=lax.Precision.HIGHEST)
    o_ref[...] = (lax.dot_general(bf, w21, (((1,), (1,)), ((), ())),
                                  preferred_element_type=jnp.float32,
                                  precision=lax.Precision.HIGHEST)
                  + r * u + b2_ref[...])


_BN = 1000


def _tc_combine(p):
    return pl.pallas_call(
        _combine_body,
        grid=(N // _BN,),
        in_specs=[pl.BlockSpec((2, _BN, DP), lambda i: (0, i, 0))],
        out_specs=pl.BlockSpec((_BN, DP), lambda i: (i, 0)),
        out_shape=jax.ShapeDtypeStruct((N, DP), jnp.float32),
    )(p)


def _tc_final(p, W1, b1, W2, b2):
    full = lambda i: (0, 0)
    return pl.pallas_call(
        _final_body,
        grid=(N // _BN,),
        in_specs=[
            pl.BlockSpec((2, _BN, DP), lambda i: (0, i, 0)),
            pl.BlockSpec((D, D), full),
            pl.BlockSpec((1, D), full),
            pl.BlockSpec((D, D), full),
            pl.BlockSpec((1, D), full),
        ],
        out_specs=pl.BlockSpec((_BN, D), lambda i: (i, 0)),
        out_shape=jax.ShapeDtypeStruct((N, D), jnp.float32),
    )(p, W1, b1, W2, b2)


@jax.jit
def kernel(x, edge_index, edge_weight, W1, b1, W2, b2):
    src = edge_index[0]
    dst = edge_index[1]
    wf = edge_weight[:, 0]
    xp = jnp.pad(x, ((0, 0), (0, DP - D)))

    p1 = _sc_agg(xp, src, dst, wf)
    a1p = _tc_combine(p1)
    p2 = _sc_agg(a1p, src, dst, wf)
    return _tc_final(p2, W1, b1.reshape(1, D), W2, b2.reshape(1, D))


# speedup vs baseline: 4.8296x; 1.2025x over previous
import functools

import jax
import jax.numpy as jnp
from jax import lax
from jax.experimental import pallas as pl
from jax.experimental.pallas import tpu as pltpu
from jax.experimental.pallas import tpu_sc as plsc

N = 10000
E = 320000
D = 128
DP = 144
NW = 32
EPT = E // NW
C = 80
CHUNKS = EPT // C
RPT = N // 16

_mesh = plsc.VectorSubcoreMesh(core_axis_name="c", subcore_axis_name="s")


@functools.partial(
    pl.kernel,
    out_type=jax.ShapeDtypeStruct((2, N, DP), jnp.float32),
    mesh=_mesh,
    scratch_types=[
        pltpu.VMEM((C,), jnp.int32),
        pltpu.VMEM((C,), jnp.int32),
        pltpu.VMEM((C,), jnp.float32),
        pltpu.VMEM((C, DP), jnp.float32),
        pltpu.VMEM((C,), jnp.int32),
        pltpu.VMEM((C,), jnp.int32),
        pltpu.VMEM((C,), jnp.float32),
        pltpu.VMEM((C, DP), jnp.float32),
        pltpu.VMEM_SHARED((N, DP), jnp.float32),
        pltpu.SemaphoreType.DMA,
        pltpu.SemaphoreType.DMA,
    ],
    compiler_params=pltpu.CompilerParams(use_tc_tiling_on_sc=False,
                                         needs_layout_passes=False),
)
def _sc_agg(x_hbm, src_hbm, dst_hbm, w_hbm, out_hbm,
            src_v, dst_v, w_v, rows_v,
            src_u, dst_u, w_u, rows_u, acc, sem0, sem1):
    cid = lax.axis_index("c")
    sid = lax.axis_index("s")
    wid = cid * 16 + sid

    zeros16 = jnp.zeros((16,), jnp.float32)

    def _zero_body(t, _):
        i = t // (DP // 16)
        j = (t % (DP // 16)) * 16
        rows_v[i, pl.ds(j, 16)] = zeros16
        return _
    lax.fori_loop(0, C * (DP // 16), _zero_body, None)

    r0 = sid * RPT
    for k in range((RPT + C - 1) // C):
        nrows = min(C, RPT - k * C)
        pltpu.sync_copy(rows_v.at[pl.ds(0, nrows)],
                        acc.at[pl.ds(r0 + k * C, nrows)])
    plsc.subcore_barrier()

    iota16 = lax.iota(jnp.int32, 16)
    ebase = wid * EPT
    bufs = ((src_v, dst_v, w_v, rows_v, sem0),
            (src_u, dst_u, w_u, rows_u, sem1))

    def _load_idx(c, b):
        base = ebase + c * C
        pltpu.sync_copy(src_hbm.at[pl.ds(base, C)], b[0])
        pltpu.sync_copy(dst_hbm.at[pl.ds(base, C)], b[1])
        pltpu.sync_copy(w_hbm.at[pl.ds(base, C)], b[2])

    def _start_gather(b):
        pltpu.async_copy(x_hbm.at[b[0]], b[3], b[4])

    def _finish(b):
        sv, dv, wv, rows, sem = b
        pltpu.make_async_copy(x_hbm.at[sv], rows, sem).wait()
        for e in range(C):
            if e % 16 == 0:
                w_blk = wv[pl.ds(e, 16)]
            w_s = jnp.sum(jnp.where(iota16 == (e % 16), w_blk, 0.0))
            wspl = jnp.full((16,), w_s, jnp.float32)
            for j in range(D // 16):
                sl = pl.ds(j * 16, 16)
                rows[e, sl] = rows[e, sl] * wspl
            ind = jnp.where(wspl != 0.0, 1.0, 0.0).astype(jnp.float32)
            sp = jnp.where(iota16 == 0, wspl,
                           jnp.where(iota16 == 1, ind, 0.0)).astype(jnp.float32)
            rows[e, pl.ds(D, 16)] = sp
        pltpu.sync_copy(rows, acc.at[dv], add=True)

    _load_idx(0, bufs[0])
    _start_gather(bufs[0])

    def _pair(i, _):
        c0 = 2 * i
        _load_idx(c0 + 1, bufs[1])
        _start_gather(bufs[1])
        _finish(bufs[0])
        _load_idx(c0 + 2, bufs[0])
        _start_gather(bufs[0])
        _finish(bufs[1])
        return _
    lax.fori_loop(0, (CHUNKS - 1) // 2, _pair, None)
    _finish(bufs[0])

    plsc.subcore_barrier()

    pltpu.sync_copy(acc.at[pl.ds(r0, RPT)], out_hbm.at[cid, pl.ds(r0, RPT)])


def _combine_body(p_ref, o_ref):
    s = p_ref[0] + p_ref[1]
    cnt = s[:, 129:130]
    o_ref[...] = s / cnt


def _final_body(p_ref, w1_ref, b1_ref, w2_ref, b2_ref, o_ref):
    s = p_ref[0] + p_ref[1]
    cnt = s[:, 129:130]
    r = s[:, 128:129] / cnt
    bf = s[:, :D] / cnt
    w21 = lax.dot_general(w2_ref[...], w1_ref[...],
                          (((1,), (0,)), ((), ())),
                          preferred_element_type=jnp.float32,
                          precision=lax.Precision.HIGHEST)
    u = lax.dot_general(b1_ref[...], w2_ref[...],
                        (((1,), (1,)), ((), ())),
                        preferred_element_type=jnp.float32,
                        precision=lax.Precision.HIGHEST)
    o_ref[...] = (lax.dot_general(bf, w21, (((1,), (1,)), ((), ())),
                                  preferred_element_type=jnp.float32,
                                  precision=lax.Precision.HIGHEST)
                  + r * u + b2_ref[...])


_BN = 1000


def _tc_combine(p):
    return pl.pallas_call(
        _combine_body,
        grid=(N // _BN,),
        in_specs=[pl.BlockSpec((2, _BN, DP), lambda i: (0, i, 0))],
        out_specs=pl.BlockSpec((_BN, DP), lambda i: (i, 0)),
        out_shape=jax.ShapeDtypeStruct((N, DP), jnp.float32),
    )(p)


def _tc_final(p, W1, b1, W2, b2):
    full = lambda i: (0, 0)
    return pl.pallas_call(
        _final_body,
        grid=(N // _BN,),
        in_specs=[
            pl.BlockSpec((2, _BN, DP), lambda i: (0, i, 0)),
            pl.BlockSpec((D, D), full),
            pl.BlockSpec((1, D), full),
            pl.BlockSpec((D, D), full),
            pl.BlockSpec((1, D), full),
        ],
        out_specs=pl.BlockSpec((_BN, D), lambda i: (i, 0)),
        out_shape=jax.ShapeDtypeStruct((N, D), jnp.float32),
    )(p, W1, b1, W2, b2)


@jax.jit
def kernel(x, edge_index, edge_weight, W1, b1, W2, b2):
    src = edge_index[0]
    dst = edge_index[1]
    wf = edge_weight[:, 0]
    xp = jnp.pad(x, ((0, 0), (0, DP - D)))

    p1 = _sc_agg(xp, src, dst, wf)
    a1p = _tc_combine(p1)
    p2 = _sc_agg(a1p, src, dst, wf)
    return _tc_final(p2, W1, b1.reshape(1, D), W2, b2.reshape(1, D))


# speedup vs baseline: 4.9744x; 1.0300x over previous
import functools

import jax
import jax.numpy as jnp
from jax import lax
from jax.experimental import pallas as pl
from jax.experimental.pallas import tpu as pltpu
from jax.experimental.pallas import tpu_sc as plsc

N = 10000
E = 320000
D = 128
DP = 144
NW = 32
EPT = E // NW
C = 80
CHUNKS = EPT // C
RPT = N // 16

_mesh = plsc.VectorSubcoreMesh(core_axis_name="c", subcore_axis_name="s")


@functools.partial(
    pl.kernel,
    out_type=jax.ShapeDtypeStruct((2, N, DP), jnp.float32),
    mesh=_mesh,
    scratch_types=[
        pltpu.VMEM((C,), jnp.int32),
        pltpu.VMEM((C,), jnp.int32),
        pltpu.VMEM((C,), jnp.float32),
        pltpu.VMEM((C, DP), jnp.float32),
        pltpu.VMEM((C,), jnp.int32),
        pltpu.VMEM((C,), jnp.int32),
        pltpu.VMEM((C,), jnp.float32),
        pltpu.VMEM((C, DP), jnp.float32),
        pltpu.VMEM_SHARED((N, DP), jnp.float32),
        pltpu.SemaphoreType.DMA,
        pltpu.SemaphoreType.DMA,
        pltpu.SemaphoreType.DMA,
        pltpu.SemaphoreType.DMA,
    ],
    compiler_params=pltpu.CompilerParams(use_tc_tiling_on_sc=False,
                                         needs_layout_passes=False),
)
def _sc_agg(x_hbm, src_hbm, dst_hbm, w_hbm, out_hbm,
            src_v, dst_v, w_v, rows_v,
            src_u, dst_u, w_u, rows_u, acc, sem0, sem1, ssem0, ssem1):
    cid = lax.axis_index("c")
    sid = lax.axis_index("s")
    wid = cid * 16 + sid

    zeros16 = jnp.zeros((16,), jnp.float32)

    def _zero_body(t, _):
        i = t // (DP // 16)
        j = (t % (DP // 16)) * 16
        rows_v[i, pl.ds(j, 16)] = zeros16
        return _
    lax.fori_loop(0, C * (DP // 16), _zero_body, None)

    r0 = sid * RPT
    for k in range((RPT + C - 1) // C):
        nrows = min(C, RPT - k * C)
        pltpu.sync_copy(rows_v.at[pl.ds(0, nrows)],
                        acc.at[pl.ds(r0 + k * C, nrows)])
    plsc.subcore_barrier()

    iota16 = lax.iota(jnp.int32, 16)
    ebase = wid * EPT
    bufs = ((src_v, dst_v, w_v, rows_v, sem0, ssem0),
            (src_u, dst_u, w_u, rows_u, sem1, ssem1))

    def _load_idx(c, b):
        base = ebase + c * C
        pltpu.sync_copy(src_hbm.at[pl.ds(base, C)], b[0])
        pltpu.sync_copy(dst_hbm.at[pl.ds(base, C)], b[1])
        pltpu.sync_copy(w_hbm.at[pl.ds(base, C)], b[2])

    def _start_gather(b):
        pltpu.async_copy(x_hbm.at[b[0]], b[3], b[4])

    def _scale(b):
        sv, dv, wv, rows, sem, ssem = b
        pltpu.make_async_copy(x_hbm.at[sv], rows, sem).wait()
        for e in range(C):
            if e % 16 == 0:
                w_blk = wv[pl.ds(e, 16)]
            w_s = jnp.sum(jnp.where(iota16 == (e % 16), w_blk, 0.0))
            wspl = jnp.full((16,), w_s, jnp.float32)
            for j in range(D // 16):
                sl = pl.ds(j * 16, 16)
                rows[e, sl] = rows[e, sl] * wspl
            ind = jnp.where(wspl != 0.0, 1.0, 0.0).astype(jnp.float32)
            sp = jnp.where(iota16 == 0, wspl,
                           jnp.where(iota16 == 1, ind, 0.0)).astype(jnp.float32)
            rows[e, pl.ds(D, 16)] = sp
        return pltpu.async_copy(rows, acc.at[dv], ssem, add=True)

    _load_idx(0, bufs[0])
    _start_gather(bufs[0])
    _load_idx(1, bufs[1])
    _start_gather(bufs[1])

    def _pair(i, _):
        c0 = 2 * i
        sc0 = _scale(bufs[0])
        sc1 = _scale(bufs[1])
        sc0.wait()
        _load_idx(c0 + 2, bufs[0])
        _start_gather(bufs[0])
        sc1.wait()
        _load_idx(c0 + 3, bufs[1])
        _start_gather(bufs[1])
        return _
    lax.fori_loop(0, (CHUNKS - 3) // 2, _pair, None)
    sc0 = _scale(bufs[0])
    sc1 = _scale(bufs[1])
    sc0.wait()
    _load_idx(CHUNKS - 1, bufs[0])
    _start_gather(bufs[0])
    sc1.wait()
    _scale(bufs[0]).wait()

    plsc.subcore_barrier()

    pltpu.sync_copy(acc.at[pl.ds(r0, RPT)], out_hbm.at[cid, pl.ds(r0, RPT)])


def _combine_body(p_ref, o_ref):
    s = p_ref[0] + p_ref[1]
    cnt = s[:, 129:130]
    o_ref[...] = s / cnt


def _final_body(p_ref, w1_ref, b1_ref, w2_ref, b2_ref, o_ref):
    s = p_ref[0] + p_ref[1]
    cnt = s[:, 129:130]
    r = s[:, 128:129] / cnt
    bf = s[:, :D] / cnt
    w21 = lax.dot_general(w2_ref[...], w1_ref[...],
                          (((1,), (0,)), ((), ())),
                          preferred_element_type=jnp.float32,
                          precision=lax.Precision.HIGHEST)
    u = lax.dot_general(b1_ref[...], w2_ref[...],
                        (((1,), (1,)), ((), ())),
                        preferred_element_type=jnp.float32,
                        precision=lax.Precision.HIGHEST)
    o_ref[...] = (lax.dot_general(bf, w21, (((1,), (1,)), ((), ())),
                                  preferred_element_type=jnp.float32,
                                  precision=lax.Precision.HIGHEST)
                  + r * u + b2_ref[...])


_BN = 1000


def _tc_combine(p):
    return pl.pallas_call(
        _combine_body,
        grid=(N // _BN,),
        in_specs=[pl.BlockSpec((2, _BN, DP), lambda i: (0, i, 0))],
        out_specs=pl.BlockSpec((_BN, DP), lambda i: (i, 0)),
        out_shape=jax.ShapeDtypeStruct((N, DP), jnp.float32),
    )(p)


def _tc_final(p, W1, b1, W2, b2):
    full = lambda i: (0, 0)
    return pl.pallas_call(
        _final_body,
        grid=(N // _BN,),
        in_specs=[
            pl.BlockSpec((2, _BN, DP), lambda i: (0, i, 0)),
            pl.BlockSpec((D, D), full),
            pl.BlockSpec((1, D), full),
            pl.BlockSpec((D, D), full),
            pl.BlockSpec((1, D), full),
        ],
        out_specs=pl.BlockSpec((_BN, D), lambda i: (i, 0)),
        out_shape=jax.ShapeDtypeStruct((N, D), jnp.float32),
    )(p, W1, b1, W2, b2)


@jax.jit
def kernel(x, edge_index, edge_weight, W1, b1, W2, b2):
    src = edge_index[0]
    dst = edge_index[1]
    wf = edge_weight[:, 0]
    xp = jnp.pad(x, ((0, 0), (0, DP - D)))

    p1 = _sc_agg(xp, src, dst, wf)
    a1p = _tc_combine(p1)
    p2 = _sc_agg(a1p, src, dst, wf)
    return _tc_final(p2, W1, b1.reshape(1, D), W2, b2.reshape(1, D))
